# baseline (speedup 1.0000x reference)
import functools

import jax
import jax.numpy as jnp
from jax import lax
from jax.experimental import pallas as pl
from jax.experimental.pallas import tpu as pltpu
from jax.experimental.pallas import tpu_sc as plsc

_NC = 2
_NS = 16
_NW = _NC * _NS
_CHUNK = 128
_NBUF = 4


@functools.lru_cache(maxsize=None)
def _build(n_tokens, vocab, embed):
    per_w = n_tokens // _NW
    n_chunks = per_w // _CHUNK
    n_tail = n_chunks % _NBUF
    main_end = n_chunks - n_tail
    assert per_w % _CHUNK == 0 and n_chunks >= _NBUF + n_tail

    mesh = plsc.VectorSubcoreMesh(core_axis_name="c", subcore_axis_name="s")

    @functools.partial(
        pl.kernel,
        out_type=jax.ShapeDtypeStruct((n_tokens, embed), jnp.float32),
        mesh=mesh,
        scratch_types=[
            pltpu.VMEM((n_chunks, _CHUNK), jnp.int32),
            [pltpu.VMEM((_CHUNK, embed), jnp.float32) for _ in range(_NBUF)],
            [pltpu.SemaphoreType.DMA for _ in range(_NBUF)],
            [pltpu.SemaphoreType.DMA for _ in range(_NBUF)],
        ],
    )
    def _emb(idx_hbm, table_hbm, out_hbm, idx_v, rows, gsem, osem):
        wid = lax.axis_index("s") * _NC + lax.axis_index("c")
        base = wid * per_w
        pltpu.sync_copy(idx_hbm.at[wid], idx_v)

        def out_at(j):
            return out_hbm.at[pl.ds(base + j * _CHUNK, _CHUNK)]

        pltpu.async_copy(table_hbm.at[idx_v.at[0]], rows[0], gsem[0])
        pltpu.async_copy(table_hbm.at[idx_v.at[1]], rows[1], gsem[1])

        @pl.loop(0, main_end, step=_NBUF)
        def _body(j):
            for b in range(_NBUF):
                jj = j + b
                b2 = (b + 2) % _NBUF
                @pl.when(jj + 2 < n_chunks)
                def _():
                    @pl.when(jj >= 2)
                    def _():
                        pltpu.make_async_copy(rows[b2], out_at(jj - 2), osem[b2]).wait()
                    pltpu.async_copy(
                        table_hbm.at[idx_v.at[jj + 2]], rows[b2], gsem[b2])
                pltpu.make_async_copy(
                    table_hbm.at[idx_v.at[jj]], rows[b], gsem[b]).wait()
                pltpu.async_copy(rows[b], out_at(jj), osem[b])

        for t in range(main_end, n_chunks):
            b = t % _NBUF
            pltpu.make_async_copy(
                table_hbm.at[idx_v.at[t]], rows[b], gsem[b]).wait()
            pltpu.async_copy(rows[b], out_at(t), osem[b])

        for t in range(n_chunks - _NBUF, n_chunks):
            pltpu.make_async_copy(rows[t % _NBUF], out_at(t), osem[t % _NBUF]).wait()

    return _emb


def kernel(x, emb_table):
    B, L = x.shape
    V, D = emb_table.shape
    n = B * L
    emb = _build(n, V, D)
    idx = x.reshape(_NW, n // (_NW * _CHUNK), _CHUNK).astype(jnp.int32)
    out = emb(idx, emb_table)
    return out.reshape(B, L, D)


# speedup vs baseline: 1.0084x; 1.0084x over previous
import functools

import jax
import jax.numpy as jnp
from jax import lax
from jax.experimental import pallas as pl
from jax.experimental.pallas import tpu as pltpu
from jax.experimental.pallas import tpu_sc as plsc

_NC = 2
_NS = 16
_NW = _NC * _NS
_CHUNK = 128
_NBUF = 6
_D = 3


@functools.lru_cache(maxsize=None)
def _build(n_tokens, vocab, embed):
    per_w = n_tokens // _NW
    n_chunks = per_w // _CHUNK
    n_tail = n_chunks % _NBUF
    main_end = n_chunks - n_tail
    assert per_w % _CHUNK == 0 and n_chunks >= _NBUF + n_tail

    mesh = plsc.VectorSubcoreMesh(core_axis_name="c", subcore_axis_name="s")

    @functools.partial(
        pl.kernel,
        out_type=jax.ShapeDtypeStruct((n_tokens, embed), jnp.float32),
        mesh=mesh,
        scratch_types=[
            pltpu.VMEM((n_chunks, _CHUNK), jnp.int32),
            [pltpu.VMEM((_CHUNK, embed), jnp.float32) for _ in range(_NBUF)],
            [pltpu.SemaphoreType.DMA for _ in range(_NBUF)],
            [pltpu.SemaphoreType.DMA for _ in range(_NBUF)],
        ],
    )
    def _emb(idx_hbm, table_hbm, out_hbm, idx_v, rows, gsem, osem):
        wid = lax.axis_index("s") * _NC + lax.axis_index("c")
        base = wid * per_w
        pltpu.sync_copy(idx_hbm.at[wid], idx_v)

        def out_at(j):
            return out_hbm.at[pl.ds(base + j * _CHUNK, _CHUNK)]

        for b in range(_D):
            pltpu.async_copy(table_hbm.at[idx_v.at[b]], rows[b], gsem[b])

        @pl.loop(0, main_end, step=_NBUF)
        def _body(j):
            for b in range(_NBUF):
                jj = j + b
                b2 = (b + _D) % _NBUF
                @pl.when(jj + _D < n_chunks)
                def _():
                    @pl.when(jj >= _NBUF - _D)
                    def _():
                        pltpu.make_async_copy(
                            rows[b2], out_at(jj + _D - _NBUF), osem[b2]).wait()
                    pltpu.async_copy(
                        table_hbm.at[idx_v.at[jj + _D]], rows[b2], gsem[b2])
                pltpu.make_async_copy(
                    table_hbm.at[idx_v.at[jj]], rows[b], gsem[b]).wait()
                pltpu.async_copy(rows[b], out_at(jj), osem[b])

        for t in range(main_end, n_chunks):
            b = t % _NBUF
            pltpu.make_async_copy(
                table_hbm.at[idx_v.at[t]], rows[b], gsem[b]).wait()
            pltpu.async_copy(rows[b], out_at(t), osem[b])

        for t in range(n_chunks - _NBUF, n_chunks):
            pltpu.make_async_copy(rows[t % _NBUF], out_at(t), osem[t % _NBUF]).wait()

    return _emb


def kernel(x, emb_table):
    B, L = x.shape
    V, D = emb_table.shape
    n = B * L
    emb = _build(n, V, D)
    idx = x.reshape(_NW, n // (_NW * _CHUNK), _CHUNK).astype(jnp.int32)
    out = emb(idx, emb_table)
    return out.reshape(B, L, D)
